# baseline (speedup 1.0000x reference)
import functools

import jax
import jax.numpy as jnp
from jax import lax
from jax.experimental import pallas as pl
from jax.experimental.pallas import tpu as pltpu
from jax.experimental.pallas import tpu_sc as plsc

D = 1024
DFF = 4096
E = 8
K = 2
N = 2048
NK = N * K

B = 768
BD = 1024
SUB = 256
NSUB = B // SUB
NS = NK // B + E
R = NS * B
NEG = -1e30


def _router_body(x_ref, wg_ref, bg_ref, pos_ref, wts_ref, meta_ref):
    x = x_ref[...]
    wg = wg_ref[...]
    logits = lax.dot_general(x, wg, (((1,), (1,)), ((), ())),
                             preferred_element_type=jnp.float32)
    logits = logits + bg_ref[...].reshape(1, E)
    iota = lax.broadcasted_iota(jnp.int32, (N, E), 1)
    v1 = jnp.max(logits, axis=1, keepdims=True)
    i1 = jnp.min(jnp.where(logits == v1, iota, E), axis=1, keepdims=True)
    masked = jnp.where(iota == i1, NEG, logits)
    v2 = jnp.max(masked, axis=1, keepdims=True)
    i2 = jnp.min(jnp.where(masked == v2, iota, E), axis=1, keepdims=True)
    t = jnp.exp(v2 - v1)
    denom = 1.0 + t
    wts_ref[...] = jnp.concatenate([1.0 / denom, t / denom], axis=1)

    oh0 = (iota == i1).astype(jnp.bfloat16)
    oh1 = (iota == i2).astype(jnp.bfloat16)
    r_io = lax.broadcasted_iota(jnp.int32, (N, N), 0)
    c_io = lax.broadcasted_iota(jnp.int32, (N, N), 1)
    tril = (r_io > c_io).astype(jnp.bfloat16)
    ohcat = jnp.concatenate([oh0, oh1], axis=1)
    pre = lax.dot_general(tril, ohcat, (((1,), (0,)), ((), ())),
                          preferred_element_type=jnp.float32)
    oh0f = oh0.astype(jnp.float32)
    oh1f = oh1.astype(jnp.float32)
    count0 = jnp.sum(oh0f, axis=0, keepdims=True)
    counts = count0 + jnp.sum(oh1f, axis=0, keepdims=True)
    steps_e = jnp.ceil(counts * (1.0 / B))
    e_r = lax.broadcasted_iota(jnp.int32, (E, E), 0)
    e_c = lax.broadcasted_iota(jnp.int32, (E, E), 1)
    sut = (e_r < e_c).astype(jnp.float32)
    step_off = lax.dot_general(steps_e, sut, (((1,), (0,)), ((), ())))
    p_off = step_off * B
    rank0 = jnp.sum(pre[:, :E] * oh0f, axis=1, keepdims=True)
    rank1 = jnp.sum((pre[:, E:] + count0) * oh1f, axis=1, keepdims=True)
    pos0 = jnp.sum(p_off * oh0f, axis=1, keepdims=True) + rank0
    pos1 = jnp.sum(p_off * oh1f, axis=1, keepdims=True) + rank1
    pos_ref[...] = jnp.concatenate([pos0, pos1], axis=1).astype(jnp.int32)

    end_off = (step_off + steps_e)
    s_io = lax.broadcasted_iota(jnp.int32, (NS, E), 0).astype(jnp.float32)
    be = jnp.sum((s_io >= end_off).astype(jnp.float32), axis=1, keepdims=True)
    be = jnp.minimum(be, float(E - 1))
    used = jnp.sum(steps_e, axis=1, keepdims=True)

    e_io = lax.broadcasted_iota(jnp.int32, (NS, E), 1).astype(jnp.float32)
    ohbe = (e_io == be).astype(jnp.float32)
    c_sel = jnp.sum(counts * ohbe, axis=1, keepdims=True)
    so_sel = jnp.sum(step_off * ohbe, axis=1, keepdims=True)
    i_col = lax.broadcasted_iota(jnp.int32, (NS, 1), 0).astype(jnp.float32)
    rows_i = jnp.clip(c_sel - (i_col - so_sel) * B, 0.0, float(B))
    subs = jnp.ceil(rows_i * (1.0 / SUB))

    meta_ref[...] = jnp.concatenate([be, used, subs], axis=0).astype(jnp.int32)


def _router(x, Wg, bg):
    return pl.pallas_call(
        _router_body,
        out_shape=(
            jax.ShapeDtypeStruct((N, K), jnp.int32),
            jax.ShapeDtypeStruct((N, K), jnp.float32),
            jax.ShapeDtypeStruct((2 * NS + 1, 1), jnp.int32),
        ),
    )(x, Wg, bg)


def _ffn_body(meta_ref, xs_ref, w1_ref, w3_ref, w2_ref, o_ref):
    i = pl.program_id(0)
    j = pl.program_id(1)
    n_act = meta_ref[NS + 1 + i]
    w1 = w1_ref[0].astype(jnp.bfloat16)
    w3 = w3_ref[0].astype(jnp.bfloat16)
    w2 = w2_ref[0].astype(jnp.bfloat16)

    def sub_body(s, _):
        rows = pl.ds(s * SUB, SUB)
        xb = xs_ref[rows, :].astype(jnp.bfloat16)
        h1 = lax.dot_general(xb, w1, (((1,), (1,)), ((), ())),
                             preferred_element_type=jnp.float32)
        h3 = lax.dot_general(xb, w3, (((1,), (1,)), ((), ())),
                             preferred_element_type=jnp.float32)
        h = h1 * (1.0 / (1.0 + jnp.exp(-h1))) * h3
        contrib = lax.dot_general(h.astype(jnp.bfloat16), w2,
                                  (((1,), (1,)), ((), ())),
                                  preferred_element_type=jnp.float32)

        @pl.when(j == 0)
        def _init():
            o_ref[rows, :] = contrib

        @pl.when(j > 0)
        def _acc():
            o_ref[rows, :] += contrib

        return 0

    lax.fori_loop(0, n_act, sub_body, 0)


def _ffn(meta, xs, W1, W3, W2):
    DFFB = DFF // BD

    def _ieff(i, m):
        return jnp.minimum(i, m[NS] - 1)

    def _jeff(i, j, m):
        return jnp.where(i < m[NS], j, DFFB - 1)

    grid_spec = pltpu.PrefetchScalarGridSpec(
        num_scalar_prefetch=1,
        grid=(NS, DFFB),
        in_specs=[
            pl.BlockSpec((B, D), lambda i, j, m: (_ieff(i, m), 0)),
            pl.BlockSpec((1, BD, D),
                         lambda i, j, m: (m[_ieff(i, m)], _jeff(i, j, m), 0)),
            pl.BlockSpec((1, BD, D),
                         lambda i, j, m: (m[_ieff(i, m)], _jeff(i, j, m), 0)),
            pl.BlockSpec((1, D, BD),
                         lambda i, j, m: (m[_ieff(i, m)], 0, _jeff(i, j, m))),
        ],
        out_specs=pl.BlockSpec((B, D), lambda i, j, m: (_ieff(i, m), 0)),
    )
    return pl.pallas_call(
        _ffn_body,
        grid_spec=grid_spec,
        out_shape=jax.ShapeDtypeStruct((R, D), jnp.float32),
    )(meta, xs, W1, W3, W2)


NW = 32
TPW = N // NW
CHUNK = 32


def _sc_dispatch(x, pos):
    mesh = plsc.VectorSubcoreMesh(core_axis_name="c", subcore_axis_name="s")

    @functools.partial(
        pl.kernel, mesh=mesh,
        out_type=jax.ShapeDtypeStruct((R, D), jnp.float32),
        scratch_types=[
            pltpu.VMEM((TPW,), jnp.int32),
            pltpu.VMEM((TPW,), jnp.int32),
            pltpu.VMEM((TPW, D), jnp.float32),
            pltpu.SemaphoreType.DMA,
        ],
    )
    def k(x_hbm, pos_hbm, xs_hbm, idx0_v, idx1_v, xbuf, sem):
        wid = lax.axis_index("s") * 2 + lax.axis_index("c")
        base = wid * TPW
        pltpu.sync_copy(pos_hbm.at[pl.ds(base, TPW)], idx0_v)
        pltpu.sync_copy(pos_hbm.at[pl.ds(N + base, TPW)], idx1_v)
        pltpu.sync_copy(x_hbm.at[pl.ds(base, TPW)], xbuf)
        cp0 = pltpu.async_copy(xbuf, xs_hbm.at[idx0_v], sem)
        cp1 = pltpu.async_copy(xbuf, xs_hbm.at[idx1_v], sem)
        cp0.wait()
        cp1.wait()

    return k(x, pos)


def _sc_combine(o, pos, w0, w1):
    mesh = plsc.VectorSubcoreMesh(core_axis_name="c", subcore_axis_name="s")

    @functools.partial(
        pl.kernel, mesh=mesh,
        out_type=jax.ShapeDtypeStruct((N, D), jnp.float32),
        scratch_types=[
            pltpu.VMEM((CHUNK,), jnp.int32),
            pltpu.VMEM((CHUNK,), jnp.int32),
            pltpu.VMEM((CHUNK,), jnp.float32),
            pltpu.VMEM((CHUNK,), jnp.float32),
            pltpu.VMEM((CHUNK, D), jnp.float32),
            pltpu.VMEM((CHUNK, D), jnp.float32),
            pltpu.SemaphoreType.DMA,
        ],
    )
    def k(o_hbm, pos_hbm, w0_hbm, w1_hbm, y_hbm,
          idx0_v, idx1_v, w0_v, w1_v, r0, r1, sem):
        wid = lax.axis_index("s") * 2 + lax.axis_index("c")
        base = wid * TPW

        def chunk_body(c, _):
            cbase = base + c * CHUNK
            pltpu.sync_copy(pos_hbm.at[pl.ds(cbase, CHUNK)], idx0_v)
            pltpu.sync_copy(pos_hbm.at[pl.ds(N + cbase, CHUNK)], idx1_v)
            pltpu.sync_copy(w0_hbm.at[pl.ds(cbase, CHUNK)], w0_v)
            pltpu.sync_copy(w1_hbm.at[pl.ds(cbase, CHUNK)], w1_v)
            cp0 = pltpu.async_copy(o_hbm.at[idx0_v], r0, sem)
            cp1 = pltpu.async_copy(o_hbm.at[idx1_v], r1, sem)
            cp0.wait()
            cp1.wait()

            def tok_body(t, _):
                lanes = jnp.zeros((16,), jnp.int32) + t
                wv0 = plsc.load_gather(w0_v, [lanes])
                wv1 = plsc.load_gather(w1_v, [lanes])

                def col_body(j, _):
                    a = r0[t, pl.ds(j * 16, 16)]
                    b = r1[t, pl.ds(j * 16, 16)]
                    r0[t, pl.ds(j * 16, 16)] = wv0 * a + wv1 * b
                    return 0

                lax.fori_loop(0, D // 16, col_body, 0)
                return 0

            lax.fori_loop(0, CHUNK, tok_body, 0)
            pltpu.sync_copy(r0, y_hbm.at[pl.ds(cbase, CHUNK)])
            return 0

        lax.fori_loop(0, TPW // CHUNK, chunk_body, 0)

    return k(o, pos, w0, w1)


def kernel(x, Wg, bg, W1, W2, W3):
    pos2, wts, meta_col = _router(x, Wg, bg)
    pos = pos2.T.reshape(-1)
    meta = meta_col.reshape(-1)

    xs = _sc_dispatch(x, pos)

    o = _ffn(meta, xs, W1, W3, W2)

    w_flat = wts.T.reshape(-1)
    y = (w_flat[:N, None] * o[pos[:N]] + w_flat[N:, None] * o[pos[N:]])
    return y


# speedup vs baseline: 1.0803x; 1.0803x over previous
import functools

import jax
import jax.numpy as jnp
from jax import lax
from jax.experimental import pallas as pl
from jax.experimental.pallas import tpu as pltpu
from jax.experimental.pallas import tpu_sc as plsc

D = 1024
DFF = 4096
E = 8
K = 2
N = 2048
NK = N * K

B = 768
BD = 1024
NS = NK // B + E
R = NS * B
NEG = -1e30


def _router_body(x_ref, wg_ref, bg_ref, pos_ref, wts_ref, meta_ref):
    x = x_ref[...]
    wg = wg_ref[...]
    logits = lax.dot_general(x, wg, (((1,), (1,)), ((), ())),
                             preferred_element_type=jnp.float32)
    logits = logits + bg_ref[...].reshape(1, E)
    iota = lax.broadcasted_iota(jnp.int32, (N, E), 1)
    v1 = jnp.max(logits, axis=1, keepdims=True)
    i1 = jnp.min(jnp.where(logits == v1, iota, E), axis=1, keepdims=True)
    masked = jnp.where(iota == i1, NEG, logits)
    v2 = jnp.max(masked, axis=1, keepdims=True)
    i2 = jnp.min(jnp.where(masked == v2, iota, E), axis=1, keepdims=True)
    t = jnp.exp(v2 - v1)
    denom = 1.0 + t
    wts_ref[...] = jnp.concatenate([1.0 / denom, t / denom], axis=1)

    oh0 = (iota == i1).astype(jnp.bfloat16)
    oh1 = (iota == i2).astype(jnp.bfloat16)
    r_io = lax.broadcasted_iota(jnp.int32, (N, N), 0)
    c_io = lax.broadcasted_iota(jnp.int32, (N, N), 1)
    tril = (r_io > c_io).astype(jnp.bfloat16)
    ohcat = jnp.concatenate([oh0, oh1], axis=1)
    pre = lax.dot_general(tril, ohcat, (((1,), (0,)), ((), ())),
                          preferred_element_type=jnp.float32)
    oh0f = oh0.astype(jnp.float32)
    oh1f = oh1.astype(jnp.float32)
    count0 = jnp.sum(oh0f, axis=0, keepdims=True)
    counts = count0 + jnp.sum(oh1f, axis=0, keepdims=True)
    steps_e = jnp.ceil(counts * (1.0 / B))
    e_r = lax.broadcasted_iota(jnp.int32, (E, E), 0)
    e_c = lax.broadcasted_iota(jnp.int32, (E, E), 1)
    sut = (e_r < e_c).astype(jnp.float32)
    step_off = lax.dot_general(steps_e, sut, (((1,), (0,)), ((), ())))
    p_off = step_off * B
    rank0 = jnp.sum(pre[:, :E] * oh0f, axis=1, keepdims=True)
    rank1 = jnp.sum((pre[:, E:] + count0) * oh1f, axis=1, keepdims=True)
    pos0 = jnp.sum(p_off * oh0f, axis=1, keepdims=True) + rank0
    pos1 = jnp.sum(p_off * oh1f, axis=1, keepdims=True) + rank1
    pos_ref[...] = jnp.concatenate([pos0, pos1], axis=1).astype(jnp.int32)

    end_off = (step_off + steps_e)
    s_io = lax.broadcasted_iota(jnp.int32, (NS, E), 0).astype(jnp.float32)
    be = jnp.sum((s_io >= end_off).astype(jnp.float32), axis=1, keepdims=True)
    be = jnp.minimum(be, float(E - 1))
    used = jnp.sum(steps_e, axis=1, keepdims=True)
    meta_ref[...] = jnp.concatenate([be, used], axis=0).astype(jnp.int32)


def _router(x, Wg, bg):
    return pl.pallas_call(
        _router_body,
        out_shape=(
            jax.ShapeDtypeStruct((N, K), jnp.int32),
            jax.ShapeDtypeStruct((N, K), jnp.float32),
            jax.ShapeDtypeStruct((NS + 1, 1), jnp.int32),
        ),
    )(x, Wg, bg)


def _ffn_body(meta_ref, xs_ref, w1_ref, w3_ref, w2_ref, o_ref):
    i = pl.program_id(0)
    j = pl.program_id(1)
    used = meta_ref[NS]

    @pl.when(i < used)
    def _compute():
        xb = xs_ref[...].astype(jnp.bfloat16)
        w1 = w1_ref[0].astype(jnp.bfloat16)
        w3 = w3_ref[0].astype(jnp.bfloat16)
        h1 = lax.dot_general(xb, w1, (((1,), (1,)), ((), ())),
                             preferred_element_type=jnp.float32)
        h3 = lax.dot_general(xb, w3, (((1,), (1,)), ((), ())),
                             preferred_element_type=jnp.float32)
        h = h1 * (1.0 / (1.0 + jnp.exp(-h1))) * h3
        contrib = lax.dot_general(h.astype(jnp.bfloat16),
                                  w2_ref[0].astype(jnp.bfloat16),
                                  (((1,), (1,)), ((), ())),
                                  preferred_element_type=jnp.float32)

        @pl.when(j == 0)
        def _init():
            o_ref[...] = contrib

        @pl.when(j > 0)
        def _acc():
            o_ref[...] += contrib


def _ffn(meta, xs, W1, W3, W2):
    DFFB = DFF // BD

    def _ieff(i, m):
        return jnp.minimum(i, m[NS] - 1)

    def _jeff(i, j, m):
        return jnp.where(i < m[NS], j, DFFB - 1)

    grid_spec = pltpu.PrefetchScalarGridSpec(
        num_scalar_prefetch=1,
        grid=(NS, DFFB),
        in_specs=[
            pl.BlockSpec((B, D), lambda i, j, m: (_ieff(i, m), 0)),
            pl.BlockSpec((1, BD, D),
                         lambda i, j, m: (m[_ieff(i, m)], _jeff(i, j, m), 0)),
            pl.BlockSpec((1, BD, D),
                         lambda i, j, m: (m[_ieff(i, m)], _jeff(i, j, m), 0)),
            pl.BlockSpec((1, D, BD),
                         lambda i, j, m: (m[_ieff(i, m)], 0, _jeff(i, j, m))),
        ],
        out_specs=pl.BlockSpec((B, D), lambda i, j, m: (_ieff(i, m), 0)),
    )
    return pl.pallas_call(
        _ffn_body,
        grid_spec=grid_spec,
        out_shape=jax.ShapeDtypeStruct((R, D), jnp.float32),
    )(meta, xs, W1, W3, W2)


NW = 32
TPW = N // NW
CHUNK = 32


def _sc_dispatch(x, pos):
    mesh = plsc.VectorSubcoreMesh(core_axis_name="c", subcore_axis_name="s")

    @functools.partial(
        pl.kernel, mesh=mesh,
        out_type=jax.ShapeDtypeStruct((R, D), jnp.float32),
        scratch_types=[
            pltpu.VMEM((TPW,), jnp.int32),
            pltpu.VMEM((TPW,), jnp.int32),
            pltpu.VMEM((TPW, D), jnp.float32),
            pltpu.SemaphoreType.DMA,
        ],
    )
    def k(x_hbm, pos_hbm, xs_hbm, idx0_v, idx1_v, xbuf, sem):
        wid = lax.axis_index("s") * 2 + lax.axis_index("c")
        base = wid * TPW
        pltpu.sync_copy(pos_hbm.at[pl.ds(base, TPW)], idx0_v)
        pltpu.sync_copy(pos_hbm.at[pl.ds(N + base, TPW)], idx1_v)
        pltpu.sync_copy(x_hbm.at[pl.ds(base, TPW)], xbuf)
        cp0 = pltpu.async_copy(xbuf, xs_hbm.at[idx0_v], sem)
        cp1 = pltpu.async_copy(xbuf, xs_hbm.at[idx1_v], sem)
        cp0.wait()
        cp1.wait()

    return k(x, pos)


def _sc_combine(o, pos, w0, w1):
    mesh = plsc.VectorSubcoreMesh(core_axis_name="c", subcore_axis_name="s")

    @functools.partial(
        pl.kernel, mesh=mesh,
        out_type=jax.ShapeDtypeStruct((N, D), jnp.float32),
        scratch_types=[
            pltpu.VMEM((CHUNK,), jnp.int32),
            pltpu.VMEM((CHUNK,), jnp.int32),
            pltpu.VMEM((CHUNK,), jnp.float32),
            pltpu.VMEM((CHUNK,), jnp.float32),
            pltpu.VMEM((CHUNK, D), jnp.float32),
            pltpu.VMEM((CHUNK, D), jnp.float32),
            pltpu.SemaphoreType.DMA,
        ],
    )
    def k(o_hbm, pos_hbm, w0_hbm, w1_hbm, y_hbm,
          idx0_v, idx1_v, w0_v, w1_v, r0, r1, sem):
        wid = lax.axis_index("s") * 2 + lax.axis_index("c")
        base = wid * TPW

        def chunk_body(c, _):
            cbase = base + c * CHUNK
            pltpu.sync_copy(pos_hbm.at[pl.ds(cbase, CHUNK)], idx0_v)
            pltpu.sync_copy(pos_hbm.at[pl.ds(N + cbase, CHUNK)], idx1_v)
            pltpu.sync_copy(w0_hbm.at[pl.ds(cbase, CHUNK)], w0_v)
            pltpu.sync_copy(w1_hbm.at[pl.ds(cbase, CHUNK)], w1_v)
            cp0 = pltpu.async_copy(o_hbm.at[idx0_v], r0, sem)
            cp1 = pltpu.async_copy(o_hbm.at[idx1_v], r1, sem)
            cp0.wait()
            cp1.wait()

            def tok_body(t, _):
                lanes = jnp.zeros((16,), jnp.int32) + t
                wv0 = plsc.load_gather(w0_v, [lanes])
                wv1 = plsc.load_gather(w1_v, [lanes])

                def col_body(j, _):
                    a = r0[t, pl.ds(j * 16, 16)]
                    b = r1[t, pl.ds(j * 16, 16)]
                    r0[t, pl.ds(j * 16, 16)] = wv0 * a + wv1 * b
                    return 0

                lax.fori_loop(0, D // 16, col_body, 0)
                return 0

            lax.fori_loop(0, CHUNK, tok_body, 0)
            pltpu.sync_copy(r0, y_hbm.at[pl.ds(cbase, CHUNK)])
            return 0

        lax.fori_loop(0, TPW // CHUNK, chunk_body, 0)

    return k(o, pos, w0, w1)


def kernel(x, Wg, bg, W1, W2, W3):
    pos2, wts, meta_col = _router(x, Wg, bg)
    pos = pos2.T.reshape(-1)
    meta = meta_col.reshape(-1)

    xs = _sc_dispatch(x, pos)

    o = _ffn(meta, xs, W1, W3, W2)

    w_flat = wts.T.reshape(-1)
    y = (w_flat[:N, None] * o[pos[:N]] + w_flat[N:, None] * o[pos[N:]])
    return y


# speedup vs baseline: 1.2253x; 1.1343x over previous
import functools

import jax
import jax.numpy as jnp
from jax import lax
from jax.experimental import pallas as pl
from jax.experimental.pallas import tpu as pltpu
from jax.experimental.pallas import tpu_sc as plsc

D = 1024
DFF = 4096
E = 8
K = 2
N = 2048
NK = N * K

B = 576
BD = 1024
NS = NK // B + E
R = NS * B
NEG = -1e30


def _router_body(x_ref, wg_ref, bg_ref, pos_ref, wts_ref, meta_ref):
    x = x_ref[...]
    wg = wg_ref[...]
    logits = lax.dot_general(x, wg, (((1,), (1,)), ((), ())),
                             preferred_element_type=jnp.float32)
    logits = logits + bg_ref[...].reshape(1, E)
    iota = lax.broadcasted_iota(jnp.int32, (N, E), 1)
    v1 = jnp.max(logits, axis=1, keepdims=True)
    i1 = jnp.min(jnp.where(logits == v1, iota, E), axis=1, keepdims=True)
    masked = jnp.where(iota == i1, NEG, logits)
    v2 = jnp.max(masked, axis=1, keepdims=True)
    i2 = jnp.min(jnp.where(masked == v2, iota, E), axis=1, keepdims=True)
    t = jnp.exp(v2 - v1)
    denom = 1.0 + t
    wts_ref[...] = jnp.concatenate([1.0 / denom, t / denom], axis=1)

    oh0 = (iota == i1).astype(jnp.bfloat16)
    oh1 = (iota == i2).astype(jnp.bfloat16)
    r_io = lax.broadcasted_iota(jnp.int32, (N, N), 0)
    c_io = lax.broadcasted_iota(jnp.int32, (N, N), 1)
    tril = (r_io > c_io).astype(jnp.bfloat16)
    ohcat = jnp.concatenate([oh0, oh1], axis=1)
    pre = lax.dot_general(tril, ohcat, (((1,), (0,)), ((), ())),
                          preferred_element_type=jnp.float32)
    oh0f = oh0.astype(jnp.float32)
    oh1f = oh1.astype(jnp.float32)
    count0 = jnp.sum(oh0f, axis=0, keepdims=True)
    counts = count0 + jnp.sum(oh1f, axis=0, keepdims=True)
    steps_e = jnp.ceil(counts * (1.0 / B))
    e_r = lax.broadcasted_iota(jnp.int32, (E, E), 0)
    e_c = lax.broadcasted_iota(jnp.int32, (E, E), 1)
    sut = (e_r < e_c).astype(jnp.float32)
    step_off = lax.dot_general(steps_e, sut, (((1,), (0,)), ((), ())))
    p_off = step_off * B
    rank0 = jnp.sum(pre[:, :E] * oh0f, axis=1, keepdims=True)
    rank1 = jnp.sum((pre[:, E:] + count0) * oh1f, axis=1, keepdims=True)
    pos0 = jnp.sum(p_off * oh0f, axis=1, keepdims=True) + rank0
    pos1 = jnp.sum(p_off * oh1f, axis=1, keepdims=True) + rank1
    pos_ref[...] = jnp.concatenate([pos0, pos1], axis=1).astype(jnp.int32)

    end_off = (step_off + steps_e)
    s_io = lax.broadcasted_iota(jnp.int32, (NS, E), 0).astype(jnp.float32)
    be = jnp.sum((s_io >= end_off).astype(jnp.float32), axis=1, keepdims=True)
    be = jnp.minimum(be, float(E - 1))
    used = jnp.sum(steps_e, axis=1, keepdims=True)
    meta_ref[...] = jnp.concatenate([be, used], axis=0).astype(jnp.int32)


def _router(x, Wg, bg):
    return pl.pallas_call(
        _router_body,
        out_shape=(
            jax.ShapeDtypeStruct((N, K), jnp.int32),
            jax.ShapeDtypeStruct((N, K), jnp.float32),
            jax.ShapeDtypeStruct((NS + 1, 1), jnp.int32),
        ),
    )(x, Wg, bg)


def _ffn_body(meta_ref, xs_ref, w1_ref, w3_ref, w2_ref, o_ref):
    i = pl.program_id(0)
    j = pl.program_id(1)
    used = meta_ref[NS]

    @pl.when(i < used)
    def _compute():
        xb = xs_ref[...].astype(jnp.bfloat16)
        w1 = w1_ref[0].astype(jnp.bfloat16)
        w3 = w3_ref[0].astype(jnp.bfloat16)
        h1 = lax.dot_general(xb, w1, (((1,), (1,)), ((), ())),
                             preferred_element_type=jnp.float32)
        h3 = lax.dot_general(xb, w3, (((1,), (1,)), ((), ())),
                             preferred_element_type=jnp.float32)
        h = h1 * (1.0 / (1.0 + jnp.exp(-h1))) * h3
        contrib = lax.dot_general(h.astype(jnp.bfloat16),
                                  w2_ref[0].astype(jnp.bfloat16),
                                  (((1,), (1,)), ((), ())),
                                  preferred_element_type=jnp.float32)

        @pl.when(j == 0)
        def _init():
            o_ref[...] = contrib

        @pl.when(j > 0)
        def _acc():
            o_ref[...] += contrib


def _ffn(meta, xs, W1, W3, W2):
    DFFB = DFF // BD

    def _ieff(i, m):
        return jnp.minimum(i, m[NS] - 1)

    def _jeff(i, j, m):
        return jnp.where(i < m[NS], j, DFFB - 1)

    grid_spec = pltpu.PrefetchScalarGridSpec(
        num_scalar_prefetch=1,
        grid=(NS, DFFB),
        in_specs=[
            pl.BlockSpec((B, D), lambda i, j, m: (_ieff(i, m), 0)),
            pl.BlockSpec((1, BD, D),
                         lambda i, j, m: (m[_ieff(i, m)], _jeff(i, j, m), 0)),
            pl.BlockSpec((1, BD, D),
                         lambda i, j, m: (m[_ieff(i, m)], _jeff(i, j, m), 0)),
            pl.BlockSpec((1, D, BD),
                         lambda i, j, m: (m[_ieff(i, m)], 0, _jeff(i, j, m))),
        ],
        out_specs=pl.BlockSpec((B, D), lambda i, j, m: (_ieff(i, m), 0)),
    )
    return pl.pallas_call(
        _ffn_body,
        grid_spec=grid_spec,
        out_shape=jax.ShapeDtypeStruct((R, D), jnp.float32),
    )(meta, xs, W1, W3, W2)


NW = 32
TPW = N // NW
CHUNK = 32


def _sc_dispatch(x, pos):
    mesh = plsc.VectorSubcoreMesh(core_axis_name="c", subcore_axis_name="s")

    @functools.partial(
        pl.kernel, mesh=mesh,
        out_type=jax.ShapeDtypeStruct((R, D), jnp.float32),
        scratch_types=[
            pltpu.VMEM((TPW,), jnp.int32),
            pltpu.VMEM((TPW,), jnp.int32),
            pltpu.VMEM((TPW, D), jnp.float32),
            pltpu.SemaphoreType.DMA,
        ],
    )
    def k(x_hbm, pos_hbm, xs_hbm, idx0_v, idx1_v, xbuf, sem):
        wid = lax.axis_index("s") * 2 + lax.axis_index("c")
        base = wid * TPW
        pltpu.sync_copy(pos_hbm.at[pl.ds(base, TPW)], idx0_v)
        pltpu.sync_copy(pos_hbm.at[pl.ds(N + base, TPW)], idx1_v)
        pltpu.sync_copy(x_hbm.at[pl.ds(base, TPW)], xbuf)
        cp0 = pltpu.async_copy(xbuf, xs_hbm.at[idx0_v], sem)
        cp1 = pltpu.async_copy(xbuf, xs_hbm.at[idx1_v], sem)
        cp0.wait()
        cp1.wait()

    return k(x, pos)


def _sc_combine(o, pos, w0, w1):
    mesh = plsc.VectorSubcoreMesh(core_axis_name="c", subcore_axis_name="s")

    @functools.partial(
        pl.kernel, mesh=mesh,
        out_type=jax.ShapeDtypeStruct((N, D), jnp.float32),
        scratch_types=[
            pltpu.VMEM((CHUNK,), jnp.int32),
            pltpu.VMEM((CHUNK,), jnp.int32),
            pltpu.VMEM((CHUNK,), jnp.float32),
            pltpu.VMEM((CHUNK,), jnp.float32),
            pltpu.VMEM((CHUNK, D), jnp.float32),
            pltpu.VMEM((CHUNK, D), jnp.float32),
            pltpu.SemaphoreType.DMA,
        ],
    )
    def k(o_hbm, pos_hbm, w0_hbm, w1_hbm, y_hbm,
          idx0_v, idx1_v, w0_v, w1_v, r0, r1, sem):
        wid = lax.axis_index("s") * 2 + lax.axis_index("c")
        base = wid * TPW

        def chunk_body(c, _):
            cbase = base + c * CHUNK
            pltpu.sync_copy(pos_hbm.at[pl.ds(cbase, CHUNK)], idx0_v)
            pltpu.sync_copy(pos_hbm.at[pl.ds(N + cbase, CHUNK)], idx1_v)
            pltpu.sync_copy(w0_hbm.at[pl.ds(cbase, CHUNK)], w0_v)
            pltpu.sync_copy(w1_hbm.at[pl.ds(cbase, CHUNK)], w1_v)
            cp0 = pltpu.async_copy(o_hbm.at[idx0_v], r0, sem)
            cp1 = pltpu.async_copy(o_hbm.at[idx1_v], r1, sem)
            cp0.wait()
            cp1.wait()

            def tok_body(t, _):
                lanes = jnp.zeros((16,), jnp.int32) + t
                wv0 = plsc.load_gather(w0_v, [lanes])
                wv1 = plsc.load_gather(w1_v, [lanes])

                def col_body(j, _):
                    a = r0[t, pl.ds(j * 16, 16)]
                    b = r1[t, pl.ds(j * 16, 16)]
                    r0[t, pl.ds(j * 16, 16)] = wv0 * a + wv1 * b
                    return 0

                lax.fori_loop(0, D // 16, col_body, 0)
                return 0

            lax.fori_loop(0, CHUNK, tok_body, 0)
            pltpu.sync_copy(r0, y_hbm.at[pl.ds(cbase, CHUNK)])
            return 0

        lax.fori_loop(0, TPW // CHUNK, chunk_body, 0)

    return k(o, pos, w0, w1)


def kernel(x, Wg, bg, W1, W2, W3):
    pos2, wts, meta_col = _router(x, Wg, bg)
    pos = pos2.T.reshape(-1)
    meta = meta_col.reshape(-1)

    xs = _sc_dispatch(x, pos)

    o = _ffn(meta, xs, W1, W3, W2)

    w_flat = wts.T.reshape(-1)
    y = (w_flat[:N, None] * o[pos[:N]] + w_flat[N:, None] * o[pos[N:]])
    return y


# speedup vs baseline: 1.2299x; 1.0037x over previous
import functools

import jax
import jax.numpy as jnp
from jax import lax
from jax.experimental import pallas as pl
from jax.experimental.pallas import tpu as pltpu
from jax.experimental.pallas import tpu_sc as plsc

D = 1024
DFF = 4096
E = 8
K = 2
N = 2048
NK = N * K

B = 576
BD = 1024
NS = NK // B + E
R = NS * B
NEG = -1e30


def _router_body(x_ref, wg_ref, bg_ref, pos_ref, wts_ref, meta_ref):
    x = x_ref[...]
    wg = wg_ref[...]
    logits = lax.dot_general(x, wg, (((1,), (1,)), ((), ())),
                             preferred_element_type=jnp.float32)
    logits = logits + bg_ref[...].reshape(1, E)
    iota = lax.broadcasted_iota(jnp.int32, (N, E), 1)
    v1 = jnp.max(logits, axis=1, keepdims=True)
    i1 = jnp.min(jnp.where(logits == v1, iota, E), axis=1, keepdims=True)
    masked = jnp.where(iota == i1, NEG, logits)
    v2 = jnp.max(masked, axis=1, keepdims=True)
    i2 = jnp.min(jnp.where(masked == v2, iota, E), axis=1, keepdims=True)
    t = jnp.exp(v2 - v1)
    denom = 1.0 + t
    wts_ref[...] = jnp.concatenate([1.0 / denom, t / denom], axis=1)

    oh0 = (iota == i1).astype(jnp.bfloat16)
    oh1 = (iota == i2).astype(jnp.bfloat16)
    r_io = lax.broadcasted_iota(jnp.int32, (N, N), 0)
    c_io = lax.broadcasted_iota(jnp.int32, (N, N), 1)
    tril = (r_io > c_io).astype(jnp.bfloat16)
    ohcat = jnp.concatenate([oh0, oh1], axis=1)
    pre = lax.dot_general(tril, ohcat, (((1,), (0,)), ((), ())),
                          preferred_element_type=jnp.float32)
    oh0f = oh0.astype(jnp.float32)
    oh1f = oh1.astype(jnp.float32)
    count0 = jnp.sum(oh0f, axis=0, keepdims=True)
    counts = count0 + jnp.sum(oh1f, axis=0, keepdims=True)
    steps_e = jnp.ceil(counts * (1.0 / B))
    e_r = lax.broadcasted_iota(jnp.int32, (E, E), 0)
    e_c = lax.broadcasted_iota(jnp.int32, (E, E), 1)
    sut = (e_r < e_c).astype(jnp.float32)
    step_off = lax.dot_general(steps_e, sut, (((1,), (0,)), ((), ())))
    p_off = step_off * B
    rank0 = jnp.sum(pre[:, :E] * oh0f, axis=1, keepdims=True)
    rank1 = jnp.sum((pre[:, E:] + count0) * oh1f, axis=1, keepdims=True)
    pos0 = jnp.sum(p_off * oh0f, axis=1, keepdims=True) + rank0
    pos1 = jnp.sum(p_off * oh1f, axis=1, keepdims=True) + rank1
    pos_ref[...] = jnp.concatenate([pos0, pos1], axis=1).astype(jnp.int32)

    end_off = (step_off + steps_e)
    s_io = lax.broadcasted_iota(jnp.int32, (NS, E), 0).astype(jnp.float32)
    be = jnp.sum((s_io >= end_off).astype(jnp.float32), axis=1, keepdims=True)
    be = jnp.minimum(be, float(E - 1))
    used = jnp.sum(steps_e, axis=1, keepdims=True)
    meta_ref[...] = jnp.concatenate([be, used], axis=0).astype(jnp.int32)


def _router(x, Wg, bg):
    return pl.pallas_call(
        _router_body,
        out_shape=(
            jax.ShapeDtypeStruct((N, K), jnp.int32),
            jax.ShapeDtypeStruct((N, K), jnp.float32),
            jax.ShapeDtypeStruct((NS + 1, 1), jnp.int32),
        ),
    )(x, Wg, bg)


def _ffn_body(meta_ref, xs_ref, w1_ref, w3_ref, w2_ref, o_ref):
    i = pl.program_id(0)
    j = pl.program_id(1)
    used = meta_ref[NS]

    @pl.when(i < used)
    def _compute():
        xb = xs_ref[...].astype(jnp.bfloat16)
        w1 = w1_ref[0].astype(jnp.bfloat16)
        w3 = w3_ref[0].astype(jnp.bfloat16)
        h1 = lax.dot_general(xb, w1, (((1,), (1,)), ((), ())),
                             preferred_element_type=jnp.float32)
        h3 = lax.dot_general(xb, w3, (((1,), (1,)), ((), ())),
                             preferred_element_type=jnp.float32)
        h = h1 * (1.0 / (1.0 + jnp.exp(-h1))) * h3
        contrib = lax.dot_general(h.astype(jnp.bfloat16),
                                  w2_ref[0].astype(jnp.bfloat16),
                                  (((1,), (1,)), ((), ())),
                                  preferred_element_type=jnp.float32)

        @pl.when(j == 0)
        def _init():
            o_ref[...] = contrib

        @pl.when(j > 0)
        def _acc():
            o_ref[...] += contrib


def _ffn(meta, xs, W1, W3, W2):
    DFFB = DFF // BD

    def _ieff(i, m):
        return jnp.minimum(i, m[NS] - 1)

    def _jeff(i, j, m):
        return jnp.where(i < m[NS], j, DFFB - 1)

    grid_spec = pltpu.PrefetchScalarGridSpec(
        num_scalar_prefetch=1,
        grid=(NS, DFFB),
        in_specs=[
            pl.BlockSpec((B, D), lambda i, j, m: (_ieff(i, m), 0)),
            pl.BlockSpec((1, BD, D),
                         lambda i, j, m: (m[_ieff(i, m)], _jeff(i, j, m), 0)),
            pl.BlockSpec((1, BD, D),
                         lambda i, j, m: (m[_ieff(i, m)], _jeff(i, j, m), 0)),
            pl.BlockSpec((1, D, BD),
                         lambda i, j, m: (m[_ieff(i, m)], 0, _jeff(i, j, m))),
        ],
        out_specs=pl.BlockSpec((B, D), lambda i, j, m: (_ieff(i, m), 0)),
    )
    return pl.pallas_call(
        _ffn_body,
        grid_spec=grid_spec,
        out_shape=jax.ShapeDtypeStruct((R, D), jnp.float32),
    )(meta, xs, W1, W3, W2)


NW = 32
TPW = N // NW


def _sc_dispatch(x, pos):
    mesh = plsc.VectorSubcoreMesh(core_axis_name="c", subcore_axis_name="s")

    @functools.partial(
        pl.kernel, mesh=mesh,
        out_type=jax.ShapeDtypeStruct((R, D), jnp.float32),
        scratch_types=[
            pltpu.VMEM((TPW,), jnp.int32),
            pltpu.VMEM((TPW,), jnp.int32),
            pltpu.VMEM((TPW, D), jnp.float32),
            pltpu.SemaphoreType.DMA,
        ],
    )
    def k(x_hbm, pos_hbm, xs_hbm, idx0_v, idx1_v, xbuf, sem):
        wid = lax.axis_index("s") * 2 + lax.axis_index("c")
        base = wid * TPW
        pltpu.sync_copy(pos_hbm.at[pl.ds(base, TPW)], idx0_v)
        pltpu.sync_copy(pos_hbm.at[pl.ds(N + base, TPW)], idx1_v)
        pltpu.sync_copy(x_hbm.at[pl.ds(base, TPW)], xbuf)
        cp0 = pltpu.async_copy(xbuf, xs_hbm.at[idx0_v], sem)
        cp1 = pltpu.async_copy(xbuf, xs_hbm.at[idx1_v], sem)
        cp0.wait()
        cp1.wait()

    return k(x, pos)


def kernel(x, Wg, bg, W1, W2, W3):
    pos2, wts, meta_col = _router(x, Wg, bg)
    pos = pos2.T.reshape(-1)
    meta = meta_col.reshape(-1)

    xs = _sc_dispatch(x, pos)

    o = _ffn(meta, xs, W1, W3, W2)

    w_flat = wts.T.reshape(-1)
    y = (w_flat[:N, None] * o[pos[:N]] + w_flat[N:, None] * o[pos[N:]])
    return y
